# Initial kernel scaffold; baseline (speedup 1.0000x reference)
import jax
import jax.numpy as jnp
from jax.experimental import pallas as pl


def kernel(input_ids, aux_features):
    raise NotImplementedError("write your pallas kernel here")


# speedup vs baseline: 2.1606x; 2.1606x over previous
import jax
import jax.numpy as jnp
from jax.experimental import pallas as pl
from jax.experimental.pallas import tpu as pltpu

_VOCAB = 6
_NAUX = 10
_H = 768
_BLK = 1024


def _body(ids_ref, aux16_ref, out_ref):
    ids = ids_ref[...]
    col = jax.lax.broadcasted_iota(jnp.int32, (_BLK, _H), 1)
    out_ref[...] = (col == ids).astype(jnp.float32)
    c16 = jax.lax.broadcasted_iota(jnp.int32, (_BLK, 16), 1)
    head = jnp.where(c16 >= _VOCAB, aux16_ref[...],
                     (c16 == ids).astype(jnp.float32))
    out_ref[:, 0:16] = head


def kernel(input_ids, aux_features):
    B, S = input_ids.shape
    N = B * S
    ids = input_ids.reshape(N, 1).astype(jnp.int32)
    aux16 = jnp.concatenate(
        [jnp.zeros((N, _VOCAB), jnp.float32),
         aux_features.reshape(N, _NAUX)], axis=1)
    grid = N // _BLK
    out = pl.pallas_call(
        _body,
        grid=(grid,),
        in_specs=[
            pl.BlockSpec((_BLK, 1), lambda i: (i, 0)),
            pl.BlockSpec((_BLK, 16), lambda i: (i, 0)),
        ],
        out_specs=pl.BlockSpec((_BLK, _H), lambda i: (i, 0)),
        out_shape=jax.ShapeDtypeStruct((N, _H), jnp.float32),
        compiler_params=pltpu.CompilerParams(
            dimension_semantics=("arbitrary",)),
    )(ids, aux16)
    return out.reshape(B, S, _H)
